# baseline (speedup 1.0000x reference)
import functools

import jax
import jax.numpy as jnp
from jax import lax
from jax.experimental import pallas as pl
from jax.experimental.pallas import tpu as pltpu
from jax.experimental.pallas import tpu_sc as plsc

_SC_NUM_CORES = 2
_SC_NUM_SUBCORES = 16


def _sc_gather(table, idx):
    B = idx.shape[0]
    nw = _SC_NUM_CORES * _SC_NUM_SUBCORES
    b_per_w = B // nw
    mesh = plsc.VectorSubcoreMesh(core_axis_name="c", subcore_axis_name="s")

    @functools.partial(
        pl.kernel,
        mesh=mesh,
        out_type=jax.ShapeDtypeStruct((B,), jnp.float32),
        scratch_types=[
            pltpu.VMEM((b_per_w,), jnp.int32),
            pltpu.VMEM((b_per_w,), jnp.float32),
            pltpu.SemaphoreType.DMA,
        ],
    )
    def gather_kernel(table_hbm, idx_hbm, out_hbm, idx_v, vals_v, sem):
        wid = lax.axis_index("s") * _SC_NUM_CORES + lax.axis_index("c")
        base = wid * b_per_w
        pltpu.sync_copy(idx_hbm.at[pl.ds(base, b_per_w)], idx_v)
        pltpu.async_copy(table_hbm.at[idx_v], vals_v, sem).wait()
        pltpu.sync_copy(vals_v, out_hbm.at[pl.ds(base, b_per_w)])

    return gather_kernel(table, idx)


_NSTREAM = 1


def _tc_main(logits_t, targets3d, cb):
    C1, B = logits_t.shape
    nblk_total = (C1 + cb - 1) // cb
    nblk = (nblk_total + _NSTREAM - 1) // _NSTREAM
    neg = -3.0e38
    last_off = (C1 - 1) - (nblk_total - 1) * cb

    def body(*refs):
        x_refs = refs[:_NSTREAM]
        t_ref, u_ref, v_ref, m_ref, s_ref, lt_ref = refs[_NSTREAM:]
        i = pl.program_id(0)
        xs = [r[...] for r in x_refs]
        rows = [
            (j * nblk + i) * cb + lax.broadcasted_iota(jnp.int32, (cb, B), 0)
            for j in range(_NSTREAM)
        ]
        xms = [jnp.where(rows[j] < C1, xs[j], neg) for j in range(_NSTREAM)]
        m_blk = functools.reduce(
            jnp.maximum,
            [jnp.max(xm, axis=0, keepdims=True) for xm in xms])
        m_prev = jnp.where(i == 0, neg, m_ref[...])
        m_new = jnp.maximum(m_prev, m_blk)
        s_blk = functools.reduce(
            jnp.add,
            [jnp.sum(jnp.exp(xm - m_new), axis=0, keepdims=True)
             for xm in xms])
        s_prev = jnp.where(i == 0, 0.0, s_ref[...])
        s_ref[...] = s_prev * jnp.exp(m_prev - m_new) + s_blk
        m_ref[...] = m_new
        lt_prev = jnp.where(i == 0, 0.0, lt_ref[...])
        lt_new = lt_prev + functools.reduce(
            jnp.add,
            [jnp.sum(jnp.where(rows[j] == t_ref[0], xs[j], 0.0),
                     axis=0, keepdims=True) for j in range(_NSTREAM)])
        lt_ref[...] = lt_new

        @pl.when(i == nblk - 1)
        def _():
            lse = jnp.log(s_ref[...]) + m_new
            lc = xs[-1][last_off:last_off + 1, :]
            u_ref[0] = lse - lt_new
            v_ref[0] = lt_new - lc

    def mk_map(j):
        return lambda i: (j * nblk + i, 0)

    return pl.pallas_call(
        body,
        grid=(nblk,),
        in_specs=[pl.BlockSpec((cb, B), mk_map(j)) for j in range(_NSTREAM)]
        + [pl.BlockSpec((1, 1, B), lambda i: (0, 0, 0))],
        out_specs=[
            pl.BlockSpec((1, 1, B), lambda i: (0, 0, 0)),
            pl.BlockSpec((1, 1, B), lambda i: (0, 0, 0)),
        ],
        out_shape=[
            jax.ShapeDtypeStruct((1, 1, B), jnp.float32),
            jax.ShapeDtypeStruct((1, 1, B), jnp.float32),
        ],
        scratch_shapes=[
            pltpu.VMEM((1, B), jnp.float32),
            pltpu.VMEM((1, B), jnp.float32),
            pltpu.VMEM((1, B), jnp.float32),
        ],
    )(*([logits_t] * _NSTREAM), targets3d)


def _tc_main_manual(logits_t, targets3d, cb):
    C1, B = logits_t.shape
    C = C1 - 1
    nblk_total = (C + cb - 1) // cb
    nblk = (nblk_total + _NSTREAM - 1) // _NSTREAM
    assert nblk == 2
    neg = -3.0e38
    last_rows = C - (nblk_total - 1) * cb

    def body(x_hbm, t_ref, u_ref, v_ref, buf, lc_buf, m_ref, s_ref, lt_ref,
             sems, sem_lc):
        i = pl.program_id(0)

        def copy(step, j, slot):
            blk = j * nblk + step
            rows_n = last_rows if blk == nblk_total - 1 else cb
            return pltpu.make_async_copy(
                x_hbm.at[pl.ds(blk * cb, rows_n), :],
                buf.at[slot, j, pl.ds(0, rows_n), :],
                sems.at[slot, j])

        def copy_lc():
            return pltpu.make_async_copy(
                x_hbm.at[pl.ds(C, 1), :], lc_buf, sem_lc)

        @pl.when(i == 0)
        def _():
            for j in range(_NSTREAM):
                copy(0, j, 0).start()
            copy_lc().start()
            for j in range(_NSTREAM):
                copy(1, j, 1).start()

        for step in range(nblk):
            @pl.when(i == step)
            def _():
                for j in range(_NSTREAM):
                    copy(step, j, step).wait()

        slot = i
        xs = [buf[slot, j] for j in range(_NSTREAM)]
        rows = [
            (j * nblk + i) * cb + lax.broadcasted_iota(jnp.int32, (cb, B), 0)
            for j in range(_NSTREAM)
        ]
        xms = [jnp.where(rows[j] < C, xs[j], neg) for j in range(_NSTREAM)]
        m_blk = functools.reduce(
            jnp.maximum,
            [jnp.max(xm, axis=0, keepdims=True) for xm in xms])
        m_prev = jnp.where(i == 0, neg, m_ref[...])
        m_new = jnp.maximum(m_prev, m_blk)
        s_blk = functools.reduce(
            jnp.add,
            [jnp.sum(jnp.exp(xm - m_new), axis=0, keepdims=True)
             for xm in xms])
        s_prev = jnp.where(i == 0, 0.0, s_ref[...])
        s_ref[...] = s_prev * jnp.exp(m_prev - m_new) + s_blk
        m_ref[...] = m_new
        lt_prev = jnp.where(i == 0, 0.0, lt_ref[...])
        lt_new = lt_prev + functools.reduce(
            jnp.add,
            [jnp.sum(jnp.where(rows[j] == t_ref[0], xs[j], 0.0),
                     axis=0, keepdims=True) for j in range(_NSTREAM)])
        lt_ref[...] = lt_new

        @pl.when(i == nblk - 1)
        def _():
            copy_lc().wait()
            lc = lc_buf[0]
            m_fin = jnp.maximum(m_new, lc)
            s_fin = s_ref[...] * jnp.exp(m_new - m_fin) + jnp.exp(lc - m_fin)
            lse = jnp.log(s_fin) + m_fin
            u_ref[0] = lse - lt_new
            v_ref[0] = lt_new - lc

    return pl.pallas_call(
        body,
        grid=(nblk,),
        in_specs=[
            pl.BlockSpec(memory_space=pltpu.MemorySpace.HBM),
            pl.BlockSpec((1, 1, B), lambda i: (0, 0, 0)),
        ],
        out_specs=[
            pl.BlockSpec((1, 1, B), lambda i: (0, 0, 0)),
            pl.BlockSpec((1, 1, B), lambda i: (0, 0, 0)),
        ],
        out_shape=[
            jax.ShapeDtypeStruct((1, 1, B), jnp.float32),
            jax.ShapeDtypeStruct((1, 1, B), jnp.float32),
        ],
        scratch_shapes=[
            pltpu.VMEM((2, _NSTREAM, cb, B), jnp.float32),
            pltpu.VMEM((1, B), jnp.float32),
            pltpu.VMEM((1, B), jnp.float32),
            pltpu.VMEM((1, B), jnp.float32),
            pltpu.VMEM((1, B), jnp.float32),
            pltpu.SemaphoreType.DMA((2, _NSTREAM)),
            pltpu.SemaphoreType.DMA,
        ],
    )(logits_t, targets3d)


def _tc_combine(u3, v3, ds3, inv_b):
    def body(u_ref, v_ref, d_ref, out_ref):
        out_ref[0, 0] = jnp.sum(
            u_ref[...] + d_ref[...] * v_ref[...]) * inv_b

    return pl.pallas_call(
        body,
        out_specs=pl.BlockSpec(memory_space=pltpu.SMEM),
        out_shape=jax.ShapeDtypeStruct((1, 1), jnp.float32),
    )(u3, v3, ds3)


def kernel(logits, targets, index, delta_smooth):
    B, _ = logits.shape
    hbm = pltpu.MemorySpace.HBM
    ds = _sc_gather(delta_smooth, index.astype(jnp.int32))
    logits_t = pltpu.with_memory_space_constraint(logits.T, hbm)
    t3 = pltpu.with_memory_space_constraint(
        targets.astype(jnp.int32).reshape(1, 1, B), hbm)
    u3, v3 = _tc_main(logits_t, t3, 512)
    out = _tc_combine(u3, v3, ds.reshape(1, 1, B), 1.0 / B)
    return out[0, 0]


# speedup vs baseline: 1.0037x; 1.0037x over previous
import functools

import jax
import jax.numpy as jnp
from jax import lax
from jax.experimental import pallas as pl
from jax.experimental.pallas import tpu as pltpu
from jax.experimental.pallas import tpu_sc as plsc

_SC_NUM_CORES = 2
_SC_NUM_SUBCORES = 16


def _sc_gather(table, idx):
    B = idx.shape[0]
    nw = _SC_NUM_CORES * _SC_NUM_SUBCORES
    b_per_w = B // nw
    mesh = plsc.VectorSubcoreMesh(core_axis_name="c", subcore_axis_name="s")

    @functools.partial(
        pl.kernel,
        mesh=mesh,
        out_type=jax.ShapeDtypeStruct((B,), jnp.float32),
        scratch_types=[
            pltpu.VMEM((b_per_w,), jnp.int32),
            pltpu.VMEM((b_per_w,), jnp.float32),
            pltpu.SemaphoreType.DMA,
        ],
    )
    def gather_kernel(table_hbm, idx_hbm, out_hbm, idx_v, vals_v, sem):
        wid = lax.axis_index("s") * _SC_NUM_CORES + lax.axis_index("c")
        base = wid * b_per_w
        pltpu.sync_copy(idx_hbm.at[pl.ds(base, b_per_w)], idx_v)
        pltpu.async_copy(table_hbm.at[idx_v], vals_v, sem).wait()
        pltpu.sync_copy(vals_v, out_hbm.at[pl.ds(base, b_per_w)])

    return gather_kernel(table, idx)


_NSTREAM = 2


def _tc_main(logits_t, targets3d, cb):
    C1, B = logits_t.shape
    nblk_total = (C1 + cb - 1) // cb
    nblk = (nblk_total + _NSTREAM - 1) // _NSTREAM
    neg = -3.0e38
    last_off = (C1 - 1) - (nblk_total - 1) * cb

    def body(*refs):
        x_refs = refs[:_NSTREAM]
        t_ref, u_ref, v_ref, m_ref, s_ref, lt_ref = refs[_NSTREAM:]
        i = pl.program_id(0)
        xs = [r[...] for r in x_refs]
        rows = [
            (j * nblk + i) * cb + lax.broadcasted_iota(jnp.int32, (cb, B), 0)
            for j in range(_NSTREAM)
        ]
        xms = [jnp.where(rows[j] < C1, xs[j], neg) for j in range(_NSTREAM)]
        m_blk = functools.reduce(
            jnp.maximum,
            [jnp.max(xm, axis=0, keepdims=True) for xm in xms])
        m_prev = jnp.where(i == 0, neg, m_ref[...])
        m_new = jnp.maximum(m_prev, m_blk)
        s_blk = functools.reduce(
            jnp.add,
            [jnp.sum(jnp.exp(xm - m_new), axis=0, keepdims=True)
             for xm in xms])
        s_prev = jnp.where(i == 0, 0.0, s_ref[...])
        s_ref[...] = s_prev * jnp.exp(m_prev - m_new) + s_blk
        m_ref[...] = m_new
        lt_prev = jnp.where(i == 0, 0.0, lt_ref[...])
        lt_new = lt_prev + functools.reduce(
            jnp.add,
            [jnp.sum(jnp.where(rows[j] == t_ref[0], xs[j], 0.0),
                     axis=0, keepdims=True) for j in range(_NSTREAM)])
        lt_ref[...] = lt_new

        @pl.when(i == nblk - 1)
        def _():
            lse = jnp.log(s_ref[...]) + m_new
            lc = xs[-1][last_off:last_off + 1, :]
            u_ref[0] = lse - lt_new
            v_ref[0] = lt_new - lc

    def mk_map(j):
        return lambda i: (j * nblk + i, 0)

    return pl.pallas_call(
        body,
        grid=(nblk,),
        in_specs=[pl.BlockSpec((cb, B), mk_map(j)) for j in range(_NSTREAM)]
        + [pl.BlockSpec((1, 1, B), lambda i: (0, 0, 0))],
        out_specs=[
            pl.BlockSpec((1, 1, B), lambda i: (0, 0, 0)),
            pl.BlockSpec((1, 1, B), lambda i: (0, 0, 0)),
        ],
        out_shape=[
            jax.ShapeDtypeStruct((1, 1, B), jnp.float32),
            jax.ShapeDtypeStruct((1, 1, B), jnp.float32),
        ],
        scratch_shapes=[
            pltpu.VMEM((1, B), jnp.float32),
            pltpu.VMEM((1, B), jnp.float32),
            pltpu.VMEM((1, B), jnp.float32),
        ],
    )(*([logits_t] * _NSTREAM), targets3d)


def _tc_main_manual(logits_t, targets3d, cb):
    C1, B = logits_t.shape
    C = C1 - 1
    nblk_total = (C + cb - 1) // cb
    nblk = (nblk_total + _NSTREAM - 1) // _NSTREAM
    assert nblk == 2
    neg = -3.0e38
    last_rows = C - (nblk_total - 1) * cb

    def body(x_hbm, t_ref, u_ref, v_ref, buf, lc_buf, m_ref, s_ref, lt_ref,
             sems, sem_lc):
        i = pl.program_id(0)

        def copy(step, j, slot):
            blk = j * nblk + step
            rows_n = last_rows if blk == nblk_total - 1 else cb
            return pltpu.make_async_copy(
                x_hbm.at[pl.ds(blk * cb, rows_n), :],
                buf.at[slot, j, pl.ds(0, rows_n), :],
                sems.at[slot, j])

        def copy_lc():
            return pltpu.make_async_copy(
                x_hbm.at[pl.ds(C, 1), :], lc_buf, sem_lc)

        @pl.when(i == 0)
        def _():
            for j in range(_NSTREAM):
                copy(0, j, 0).start()
            copy_lc().start()
            for j in range(_NSTREAM):
                copy(1, j, 1).start()

        for step in range(nblk):
            @pl.when(i == step)
            def _():
                for j in range(_NSTREAM):
                    copy(step, j, step).wait()

        slot = i
        xs = [buf[slot, j] for j in range(_NSTREAM)]
        rows = [
            (j * nblk + i) * cb + lax.broadcasted_iota(jnp.int32, (cb, B), 0)
            for j in range(_NSTREAM)
        ]
        xms = [jnp.where(rows[j] < C, xs[j], neg) for j in range(_NSTREAM)]
        m_blk = functools.reduce(
            jnp.maximum,
            [jnp.max(xm, axis=0, keepdims=True) for xm in xms])
        m_prev = jnp.where(i == 0, neg, m_ref[...])
        m_new = jnp.maximum(m_prev, m_blk)
        s_blk = functools.reduce(
            jnp.add,
            [jnp.sum(jnp.exp(xm - m_new), axis=0, keepdims=True)
             for xm in xms])
        s_prev = jnp.where(i == 0, 0.0, s_ref[...])
        s_ref[...] = s_prev * jnp.exp(m_prev - m_new) + s_blk
        m_ref[...] = m_new
        lt_prev = jnp.where(i == 0, 0.0, lt_ref[...])
        lt_new = lt_prev + functools.reduce(
            jnp.add,
            [jnp.sum(jnp.where(rows[j] == t_ref[0], xs[j], 0.0),
                     axis=0, keepdims=True) for j in range(_NSTREAM)])
        lt_ref[...] = lt_new

        @pl.when(i == nblk - 1)
        def _():
            copy_lc().wait()
            lc = lc_buf[0]
            m_fin = jnp.maximum(m_new, lc)
            s_fin = s_ref[...] * jnp.exp(m_new - m_fin) + jnp.exp(lc - m_fin)
            lse = jnp.log(s_fin) + m_fin
            u_ref[0] = lse - lt_new
            v_ref[0] = lt_new - lc

    return pl.pallas_call(
        body,
        grid=(nblk,),
        in_specs=[
            pl.BlockSpec(memory_space=pltpu.MemorySpace.HBM),
            pl.BlockSpec((1, 1, B), lambda i: (0, 0, 0)),
        ],
        out_specs=[
            pl.BlockSpec((1, 1, B), lambda i: (0, 0, 0)),
            pl.BlockSpec((1, 1, B), lambda i: (0, 0, 0)),
        ],
        out_shape=[
            jax.ShapeDtypeStruct((1, 1, B), jnp.float32),
            jax.ShapeDtypeStruct((1, 1, B), jnp.float32),
        ],
        scratch_shapes=[
            pltpu.VMEM((2, _NSTREAM, cb, B), jnp.float32),
            pltpu.VMEM((1, B), jnp.float32),
            pltpu.VMEM((1, B), jnp.float32),
            pltpu.VMEM((1, B), jnp.float32),
            pltpu.VMEM((1, B), jnp.float32),
            pltpu.SemaphoreType.DMA((2, _NSTREAM)),
            pltpu.SemaphoreType.DMA,
        ],
    )(logits_t, targets3d)


def _tc_combine(u3, v3, ds3, inv_b):
    def body(u_ref, v_ref, d_ref, out_ref):
        out_ref[0, 0] = jnp.sum(
            u_ref[...] + d_ref[...] * v_ref[...]) * inv_b

    return pl.pallas_call(
        body,
        out_specs=pl.BlockSpec(memory_space=pltpu.SMEM),
        out_shape=jax.ShapeDtypeStruct((1, 1), jnp.float32),
    )(u3, v3, ds3)


def kernel(logits, targets, index, delta_smooth):
    B, _ = logits.shape
    hbm = pltpu.MemorySpace.HBM
    ds = _sc_gather(delta_smooth, index.astype(jnp.int32))
    logits_t = pltpu.with_memory_space_constraint(logits.T, hbm)
    t3 = pltpu.with_memory_space_constraint(
        targets.astype(jnp.int32).reshape(1, 1, B), hbm)
    u3, v3 = _tc_main(logits_t, t3, 256)
    out = _tc_combine(u3, v3, ds.reshape(1, 1, B), 1.0 / B)
    return out[0, 0]


# speedup vs baseline: 1.0290x; 1.0252x over previous
import functools

import jax
import jax.numpy as jnp
from jax import lax
from jax.experimental import pallas as pl
from jax.experimental.pallas import tpu as pltpu
from jax.experimental.pallas import tpu_sc as plsc

_SC_NUM_CORES = 2
_SC_NUM_SUBCORES = 16


def _sc_gather(table, idx):
    B = idx.shape[0]
    nw = _SC_NUM_CORES * _SC_NUM_SUBCORES
    b_per_w = B // nw
    mesh = plsc.VectorSubcoreMesh(core_axis_name="c", subcore_axis_name="s")

    @functools.partial(
        pl.kernel,
        mesh=mesh,
        out_type=jax.ShapeDtypeStruct((B,), jnp.float32),
        scratch_types=[
            pltpu.VMEM((b_per_w,), jnp.int32),
            pltpu.VMEM((b_per_w,), jnp.float32),
            pltpu.SemaphoreType.DMA,
        ],
    )
    def gather_kernel(table_hbm, idx_hbm, out_hbm, idx_v, vals_v, sem):
        wid = lax.axis_index("s") * _SC_NUM_CORES + lax.axis_index("c")
        base = wid * b_per_w
        pltpu.sync_copy(idx_hbm.at[pl.ds(base, b_per_w)], idx_v)
        pltpu.async_copy(table_hbm.at[idx_v], vals_v, sem).wait()
        pltpu.sync_copy(vals_v, out_hbm.at[pl.ds(base, b_per_w)])

    return gather_kernel(table, idx)


_NSTREAM = 1


def _tc_main(logits_t, targets3d, cb):
    C1, B = logits_t.shape
    nblk_total = (C1 + cb - 1) // cb
    nblk = (nblk_total + _NSTREAM - 1) // _NSTREAM
    neg = -3.0e38
    last_off = (C1 - 1) - (nblk_total - 1) * cb

    def body(*refs):
        x_refs = refs[:_NSTREAM]
        t_ref, u_ref, v_ref, m_ref, s_ref, lt_ref = refs[_NSTREAM:]
        i = pl.program_id(0)
        xs = [r[...] for r in x_refs]
        rows = [
            (j * nblk + i) * cb + lax.broadcasted_iota(jnp.int32, (cb, B), 0)
            for j in range(_NSTREAM)
        ]
        xms = [jnp.where(rows[j] < C1, xs[j], neg) for j in range(_NSTREAM)]
        m_blk = functools.reduce(
            jnp.maximum,
            [jnp.max(xm, axis=0, keepdims=True) for xm in xms])
        m_prev = jnp.where(i == 0, neg, m_ref[...])
        m_new = jnp.maximum(m_prev, m_blk)
        s_blk = functools.reduce(
            jnp.add,
            [jnp.sum(jnp.exp(xm - m_new), axis=0, keepdims=True)
             for xm in xms])
        s_prev = jnp.where(i == 0, 0.0, s_ref[...])
        s_ref[...] = s_prev * jnp.exp(m_prev - m_new) + s_blk
        m_ref[...] = m_new
        lt_prev = jnp.where(i == 0, 0.0, lt_ref[...])
        lt_new = lt_prev + functools.reduce(
            jnp.add,
            [jnp.sum(jnp.where(rows[j] == t_ref[0], xs[j], 0.0),
                     axis=0, keepdims=True) for j in range(_NSTREAM)])
        lt_ref[...] = lt_new

        @pl.when(i == nblk - 1)
        def _():
            lse = jnp.log(s_ref[...]) + m_new
            lc = xs[-1][last_off:last_off + 1, :]
            u_ref[0] = lse - lt_new
            v_ref[0] = lt_new - lc

    def mk_map(j):
        return lambda i: (j * nblk + i, 0)

    return pl.pallas_call(
        body,
        grid=(nblk,),
        in_specs=[pl.BlockSpec((cb, B), mk_map(j)) for j in range(_NSTREAM)]
        + [pl.BlockSpec((1, 1, B), lambda i: (0, 0, 0))],
        out_specs=[
            pl.BlockSpec((1, 1, B), lambda i: (0, 0, 0)),
            pl.BlockSpec((1, 1, B), lambda i: (0, 0, 0)),
        ],
        out_shape=[
            jax.ShapeDtypeStruct((1, 1, B), jnp.float32),
            jax.ShapeDtypeStruct((1, 1, B), jnp.float32),
        ],
        scratch_shapes=[
            pltpu.VMEM((1, B), jnp.float32),
            pltpu.VMEM((1, B), jnp.float32),
            pltpu.VMEM((1, B), jnp.float32),
        ],
    )(*([logits_t] * _NSTREAM), targets3d)


def _tc_main_manual(logits_t, targets3d, cb):
    C1, B = logits_t.shape
    C = C1 - 1
    nblk_total = (C + cb - 1) // cb
    nblk = (nblk_total + _NSTREAM - 1) // _NSTREAM
    assert nblk == 2
    neg = -3.0e38
    last_rows = C - (nblk_total - 1) * cb

    def body(x_hbm, t_ref, u_ref, v_ref, buf, lc_buf, m_ref, s_ref, lt_ref,
             sems, sem_lc):
        i = pl.program_id(0)

        def copy(step, j, slot):
            blk = j * nblk + step
            rows_n = last_rows if blk == nblk_total - 1 else cb
            return pltpu.make_async_copy(
                x_hbm.at[pl.ds(blk * cb, rows_n), :],
                buf.at[slot, j, pl.ds(0, rows_n), :],
                sems.at[slot, j])

        def copy_lc():
            return pltpu.make_async_copy(
                x_hbm.at[pl.ds(C, 1), :], lc_buf, sem_lc)

        @pl.when(i == 0)
        def _():
            for j in range(_NSTREAM):
                copy(0, j, 0).start()
            copy_lc().start()
            for j in range(_NSTREAM):
                copy(1, j, 1).start()

        for step in range(nblk):
            @pl.when(i == step)
            def _():
                for j in range(_NSTREAM):
                    copy(step, j, step).wait()

        slot = i
        xs = [buf[slot, j] for j in range(_NSTREAM)]
        rows = [
            (j * nblk + i) * cb + lax.broadcasted_iota(jnp.int32, (cb, B), 0)
            for j in range(_NSTREAM)
        ]
        xms = [jnp.where(rows[j] < C, xs[j], neg) for j in range(_NSTREAM)]
        m_blk = functools.reduce(
            jnp.maximum,
            [jnp.max(xm, axis=0, keepdims=True) for xm in xms])
        m_prev = jnp.where(i == 0, neg, m_ref[...])
        m_new = jnp.maximum(m_prev, m_blk)
        s_blk = functools.reduce(
            jnp.add,
            [jnp.sum(jnp.exp(xm - m_new), axis=0, keepdims=True)
             for xm in xms])
        s_prev = jnp.where(i == 0, 0.0, s_ref[...])
        s_ref[...] = s_prev * jnp.exp(m_prev - m_new) + s_blk
        m_ref[...] = m_new
        lt_prev = jnp.where(i == 0, 0.0, lt_ref[...])
        lt_new = lt_prev + functools.reduce(
            jnp.add,
            [jnp.sum(jnp.where(rows[j] == t_ref[0], xs[j], 0.0),
                     axis=0, keepdims=True) for j in range(_NSTREAM)])
        lt_ref[...] = lt_new

        @pl.when(i == nblk - 1)
        def _():
            copy_lc().wait()
            lc = lc_buf[0]
            m_fin = jnp.maximum(m_new, lc)
            s_fin = s_ref[...] * jnp.exp(m_new - m_fin) + jnp.exp(lc - m_fin)
            lse = jnp.log(s_fin) + m_fin
            u_ref[0] = lse - lt_new
            v_ref[0] = lt_new - lc

    return pl.pallas_call(
        body,
        grid=(nblk,),
        in_specs=[
            pl.BlockSpec(memory_space=pltpu.MemorySpace.HBM),
            pl.BlockSpec((1, 1, B), lambda i: (0, 0, 0)),
        ],
        out_specs=[
            pl.BlockSpec((1, 1, B), lambda i: (0, 0, 0)),
            pl.BlockSpec((1, 1, B), lambda i: (0, 0, 0)),
        ],
        out_shape=[
            jax.ShapeDtypeStruct((1, 1, B), jnp.float32),
            jax.ShapeDtypeStruct((1, 1, B), jnp.float32),
        ],
        scratch_shapes=[
            pltpu.VMEM((2, _NSTREAM, cb, B), jnp.float32),
            pltpu.VMEM((1, B), jnp.float32),
            pltpu.VMEM((1, B), jnp.float32),
            pltpu.VMEM((1, B), jnp.float32),
            pltpu.VMEM((1, B), jnp.float32),
            pltpu.SemaphoreType.DMA((2, _NSTREAM)),
            pltpu.SemaphoreType.DMA,
        ],
    )(logits_t, targets3d)


def _tc_combine(u3, v3, ds3, inv_b):
    def body(u_ref, v_ref, d_ref, out_ref):
        out_ref[0, 0] = jnp.sum(
            u_ref[...] + d_ref[...] * v_ref[...]) * inv_b

    return pl.pallas_call(
        body,
        out_specs=pl.BlockSpec(memory_space=pltpu.SMEM),
        out_shape=jax.ShapeDtypeStruct((1, 1), jnp.float32),
    )(u3, v3, ds3)


def kernel(logits, targets, index, delta_smooth):
    B, _ = logits.shape
    hbm = pltpu.MemorySpace.HBM
    ds = _sc_gather(delta_smooth, index.astype(jnp.int32))
    logits_t = pltpu.with_memory_space_constraint(logits.T, hbm)
    t3 = pltpu.with_memory_space_constraint(
        targets.astype(jnp.int32).reshape(1, 1, B), hbm)
    u3, v3 = _tc_main(logits_t, t3, 336)
    out = _tc_combine(u3, v3, ds.reshape(1, 1, B), 1.0 / B)
    return out[0, 0]


# speedup vs baseline: 1.0331x; 1.0039x over previous
import functools

import jax
import jax.numpy as jnp
from jax import lax
from jax.experimental import pallas as pl
from jax.experimental.pallas import tpu as pltpu
from jax.experimental.pallas import tpu_sc as plsc

_SC_NUM_CORES = 2
_SC_NUM_SUBCORES = 16


def _sc_gather(table, idx):
    B = idx.shape[0]
    nw = _SC_NUM_CORES * _SC_NUM_SUBCORES
    b_per_w = B // nw
    mesh = plsc.VectorSubcoreMesh(core_axis_name="c", subcore_axis_name="s")

    @functools.partial(
        pl.kernel,
        mesh=mesh,
        out_type=jax.ShapeDtypeStruct((B,), jnp.float32),
        scratch_types=[
            pltpu.VMEM((b_per_w,), jnp.int32),
            pltpu.VMEM((b_per_w,), jnp.float32),
            pltpu.SemaphoreType.DMA,
        ],
    )
    def gather_kernel(table_hbm, idx_hbm, out_hbm, idx_v, vals_v, sem):
        wid = lax.axis_index("s") * _SC_NUM_CORES + lax.axis_index("c")
        base = wid * b_per_w
        pltpu.sync_copy(idx_hbm.at[pl.ds(base, b_per_w)], idx_v)
        pltpu.async_copy(table_hbm.at[idx_v], vals_v, sem).wait()
        pltpu.sync_copy(vals_v, out_hbm.at[pl.ds(base, b_per_w)])

    return gather_kernel(table, idx)


_NSTREAM = 1


def _tc_main(logits_t, targets3d, cb):
    C1, B = logits_t.shape
    nblk_total = (C1 + cb - 1) // cb
    nblk = (nblk_total + _NSTREAM - 1) // _NSTREAM
    neg = -3.0e38
    last_off = (C1 - 1) - (nblk_total - 1) * cb

    def body(*refs):
        x_refs = refs[:_NSTREAM]
        t_ref, u_ref, v_ref, m_ref, s_ref, lt_ref = refs[_NSTREAM:]
        i = pl.program_id(0)
        xs = [r[...] for r in x_refs]
        rows = [
            (j * nblk + i) * cb + lax.broadcasted_iota(jnp.int32, (cb, B), 0)
            for j in range(_NSTREAM)
        ]
        xms = [jnp.where(rows[j] < C1, xs[j], neg) for j in range(_NSTREAM)]
        m_blk = functools.reduce(
            jnp.maximum,
            [jnp.max(xm, axis=0, keepdims=True) for xm in xms])
        m_prev = jnp.where(i == 0, neg, m_ref[...])
        m_new = jnp.maximum(m_prev, m_blk)
        s_blk = functools.reduce(
            jnp.add,
            [jnp.sum(jnp.exp(xm - m_new), axis=0, keepdims=True)
             for xm in xms])
        s_prev = jnp.where(i == 0, 0.0, s_ref[...])
        s_ref[...] = s_prev * jnp.exp(m_prev - m_new) + s_blk
        m_ref[...] = m_new
        lt_prev = jnp.where(i == 0, 0.0, lt_ref[...])
        lt_new = lt_prev + functools.reduce(
            jnp.add,
            [jnp.sum(jnp.where(rows[j] == t_ref[0], xs[j], 0.0),
                     axis=0, keepdims=True) for j in range(_NSTREAM)])
        lt_ref[...] = lt_new

        @pl.when(i == nblk - 1)
        def _():
            lse = jnp.log(s_ref[...]) + m_new
            lc = xs[-1][last_off:last_off + 1, :]
            u_ref[0] = lse - lt_new
            v_ref[0] = lt_new - lc

    def mk_map(j):
        return lambda i: (j * nblk + i, 0)

    return pl.pallas_call(
        body,
        grid=(nblk,),
        in_specs=[pl.BlockSpec((cb, B), mk_map(j)) for j in range(_NSTREAM)]
        + [pl.BlockSpec((1, 1, B), lambda i: (0, 0, 0))],
        out_specs=[
            pl.BlockSpec((1, 1, B), lambda i: (0, 0, 0)),
            pl.BlockSpec((1, 1, B), lambda i: (0, 0, 0)),
        ],
        out_shape=[
            jax.ShapeDtypeStruct((1, 1, B), jnp.float32),
            jax.ShapeDtypeStruct((1, 1, B), jnp.float32),
        ],
        scratch_shapes=[
            pltpu.VMEM((1, B), jnp.float32),
            pltpu.VMEM((1, B), jnp.float32),
            pltpu.VMEM((1, B), jnp.float32),
        ],
    )(*([logits_t] * _NSTREAM), targets3d)


def _tc_combine(u3, v3, ds3, inv_b):
    def body(u_ref, v_ref, d_ref, out_ref):
        out_ref[0, 0] = jnp.sum(
            u_ref[...] + d_ref[...] * v_ref[...]) * inv_b

    return pl.pallas_call(
        body,
        out_specs=pl.BlockSpec(memory_space=pltpu.SMEM),
        out_shape=jax.ShapeDtypeStruct((1, 1), jnp.float32),
    )(u3, v3, ds3)


def kernel(logits, targets, index, delta_smooth):
    B, _ = logits.shape
    hbm = pltpu.MemorySpace.HBM
    ds = _sc_gather(delta_smooth, index.astype(jnp.int32))
    logits_t = pltpu.with_memory_space_constraint(logits.T, hbm)
    t3 = pltpu.with_memory_space_constraint(
        targets.astype(jnp.int32).reshape(1, 1, B), hbm)
    u3, v3 = _tc_main(logits_t, t3, 256)
    out = _tc_combine(u3, v3, ds.reshape(1, 1, B), 1.0 / B)
    return out[0, 0]
